# baseline (speedup 1.0000x reference)
import jax
import jax.numpy as jnp
from jax.experimental import pallas as pl

_G = 2


def _body(x_ref, wg_ref, bg_ref, wn_ref, bn_ref, o_ref):
    n = x_ref.shape[0] // _G
    gates = jax.lax.dot_general(
        wg_ref[...], x_ref[...], (((1,), (1,)), ((), ())),
        preferred_element_type=jnp.float32)
    ms = [jnp.max(gates[:, g * n:(g + 1) * n]) for g in range(_G)]
    cols = jax.lax.broadcasted_iota(jnp.int32, (1, _G * n), 1)
    gcol = cols // n
    mvec = ms[_G - 1]
    for g in range(_G - 2, -1, -1):
        mvec = jnp.where(gcol == g, ms[g], mvec)
    e = jnp.exp(gates - mvec)
    ss = [jnp.sum(e[:, g * n:(g + 1) * n]) for g in range(_G)]
    rows = jax.lax.broadcasted_iota(jnp.int32, (_G, _G * n), 0)
    emat = jnp.where(rows == (cols // n),
                     jnp.broadcast_to(e, (_G, _G * n)), 0.0)
    pcat = jnp.dot(emat, x_ref[...],
                   preferred_element_type=jnp.float32)
    out = jnp.dot(pcat, wn_ref[...],
                  preferred_element_type=jnp.float32)
    for g in range(_G):
        inv = 1.0 / (ss[g] + 1e-16)
        o_ref[g] = out[g:g + 1, :] * inv + bn_ref[...] * (ss[g] * inv)


def kernel(x, Wg, bg, Wn, bn):
    bz, n, f = x.shape
    xf = x.reshape(bz * n, f)
    wgT = Wg.reshape(1, f)
    bg2 = bg.reshape(1, 1)
    bn2 = bn.reshape(1, f)
    nb = bz // _G
    return pl.pallas_call(
        _body,
        grid=(nb,),
        in_specs=[
            pl.BlockSpec((_G * n, f), lambda b: (b, 0)),
            pl.BlockSpec((1, f), lambda b: (0, 0)),
            pl.BlockSpec((1, 1), lambda b: (0, 0)),
            pl.BlockSpec((f, f), lambda b: (0, 0)),
            pl.BlockSpec((1, f), lambda b: (0, 0)),
        ],
        out_specs=pl.BlockSpec((_G, 1, f), lambda b: (b, 0, 0)),
        out_shape=jax.ShapeDtypeStruct((bz, 1, f), jnp.float32),
    )(xf, wgT, bg2, Wn, bn2).reshape(bz, f)


# speedup vs baseline: 1.0291x; 1.0291x over previous
import jax
import jax.numpy as jnp
from jax.experimental import pallas as pl

_G = 2


def _body(x_ref, wg_ref, bg_ref, wn_ref, bn_ref, o_ref):
    n = x_ref.shape[0] // _G
    gates = jax.lax.dot_general(
        wg_ref[...], x_ref[...], (((1,), (1,)), ((), ())),
        preferred_element_type=jnp.float32)
    pooled = []
    scales = []
    for g in range(_G):
        xb = x_ref[g * n:(g + 1) * n, :]
        gate = gates[:, g * n:(g + 1) * n]
        m = jnp.max(gate)
        e = jnp.exp(gate - m)
        s = jnp.sum(e)
        p = jnp.dot(e, xb, preferred_element_type=jnp.float32)
        inv = 1.0 / (s + 1e-16)
        pooled.append(p * inv)
        scales.append(s * inv)
    pcat = jnp.concatenate(pooled, axis=0)
    out = jnp.dot(pcat, wn_ref[...],
                  preferred_element_type=jnp.float32)
    for g in range(_G):
        o_ref[g] = out[g:g + 1, :] + bn_ref[...] * scales[g]


def kernel(x, Wg, bg, Wn, bn):
    bz, n, f = x.shape
    xf = x.reshape(bz * n, f)
    wgT = Wg.reshape(1, f)
    bg2 = bg.reshape(1, 1)
    bn2 = bn.reshape(1, f)
    nb = bz // _G
    return pl.pallas_call(
        _body,
        grid=(nb,),
        in_specs=[
            pl.BlockSpec((_G * n, f), lambda b: (b, 0)),
            pl.BlockSpec((1, f), lambda b: (0, 0)),
            pl.BlockSpec((1, 1), lambda b: (0, 0)),
            pl.BlockSpec((f, f), lambda b: (0, 0)),
            pl.BlockSpec((1, f), lambda b: (0, 0)),
        ],
        out_specs=pl.BlockSpec((_G, 1, f), lambda b: (b, 0, 0)),
        out_shape=jax.ShapeDtypeStruct((bz, 1, f), jnp.float32),
    )(xf, wgT, bg2, Wn, bn2).reshape(bz, f)
